# baseline (speedup 1.0000x reference)
import functools

import jax
import jax.numpy as jnp
from jax import lax
from jax.experimental import pallas as pl
from jax.experimental.pallas import tpu as pltpu
from jax.experimental.pallas import tpu_sc as plsc

VOCAB = 1000000
D = 64
CTX = 200

NC = 2
NS = 16
NW = NC * NS
ROWS_PER_W = 8
CTX_PAD = (NW + 1) * ROWS_PER_W
ACTIVE_W = CTX // ROWS_PER_W

BN = 32768


def _sc_gather_mean(ctx_pad, in_emb_flat):
    mesh = plsc.VectorSubcoreMesh(core_axis_name="c", subcore_axis_name="s")

    @functools.partial(
        pl.kernel,
        out_type=jax.ShapeDtypeStruct((NW, D), jnp.float32),
        mesh=mesh,
        scratch_types=[
            pltpu.VMEM((16,), jnp.int32),
            pltpu.VMEM((ROWS_PER_W, D), jnp.int32),
            pltpu.VMEM((ROWS_PER_W, D), jnp.float32),
            pltpu.VMEM((D,), jnp.float32),
            pltpu.SemaphoreType.DMA,
        ],
        compiler_params=pltpu.CompilerParams(needs_layout_passes=False),
    )
    def k(ctx_hbm, emb_hbm, out_hbm, idx_v, pos_v, rows_v, acc_v, sem):
        wid = lax.axis_index("s") * NC + lax.axis_index("c")
        base = wid * ROWS_PER_W
        pltpu.sync_copy(ctx_hbm.at[pl.ds(base, 16)], idx_v)
        idx_vec = idx_v[...]
        lanes = lax.iota(jnp.int32, 16)
        copies = []
        for j in range(ROWS_PER_W):
            ij = jnp.sum(idx_vec * (lanes == j).astype(jnp.int32))
            for c in range(D // 16):
                ramp = (lax.iota(jnp.int32, 16) + (c * 16)) * VOCAB
                pos_v[j, pl.ds(c * 16, 16)] = ramp + ij
            copies.append(
                pltpu.async_copy(emb_hbm.at[pos_v.at[j]], rows_v.at[j], sem)
            )
        for cp in copies:
            cp.wait()
        scale = jnp.where(wid < ACTIVE_W, jnp.float32(1.0 / CTX), jnp.float32(0.0))
        for c in range(D // 16):
            s = rows_v[0, pl.ds(c * 16, 16)]
            for i in range(1, ROWS_PER_W):
                s = s + rows_v[i, pl.ds(c * 16, 16)]
            acc_v[pl.ds(c * 16, 16)] = s * scale
        pltpu.sync_copy(acc_v, out_hbm.at[wid])

    return k(ctx_pad, in_emb_flat)


def _tc_matvec(partials, out_emb_t):
    grid = pl.cdiv(VOCAB, BN)

    def body(part_ref, emb_ref, out_ref):
        v = jnp.sum(part_ref[...], axis=0).reshape(1, D)
        out_ref[...] = jax.lax.dot_general(
            v, emb_ref[...], (((1,), (0,)), ((), ())),
            preferred_element_type=jnp.float32).reshape(1, 1, BN)

    out2 = pl.pallas_call(
        body,
        grid=(grid,),
        in_specs=[
            pl.BlockSpec((NW, D), lambda i: (0, 0)),
            pl.BlockSpec((D, BN), lambda i: (0, i)),
        ],
        out_specs=pl.BlockSpec((1, 1, BN), lambda i: (i, 0, 0)),
        out_shape=jax.ShapeDtypeStruct((grid, 1, BN), jnp.float32),
    )(partials, out_emb_t)
    return out2.reshape(-1)[:VOCAB]


def kernel(context, in_emb, out_emb):
    ctx_pad = jnp.zeros((CTX_PAD,), jnp.int32).at[:CTX].set(context.astype(jnp.int32))
    partials = _sc_gather_mean(ctx_pad, in_emb.T.reshape(-1))
    return _tc_matvec(partials, out_emb.T)


# speedup vs baseline: 46.2963x; 46.2963x over previous
import functools

import jax
import jax.numpy as jnp
from jax import lax
from jax.experimental import pallas as pl
from jax.experimental.pallas import tpu as pltpu
from jax.experimental.pallas import tpu_sc as plsc

VOCAB = 1000000
D = 64
CTX = 200

NC = 2
NS = 16
NW = NC * NS
ROWS_PER_W = 8
CTX_PAD = (NW + 1) * ROWS_PER_W
ACTIVE_W = CTX // ROWS_PER_W

BN = 32768


def _sc_gather_mean(ctx_pad, in_emb_t):
    mesh = plsc.VectorSubcoreMesh(core_axis_name="c", subcore_axis_name="s")

    @functools.partial(
        pl.kernel,
        out_type=jax.ShapeDtypeStruct((NW, D), jnp.float32),
        mesh=mesh,
        scratch_types=[
            pltpu.VMEM((16,), jnp.int32),
            pltpu.VMEM((ROWS_PER_W, D, 128), jnp.float32),
            pltpu.VMEM((D,), jnp.float32),
            pltpu.SemaphoreType.DMA,
        ],
        compiler_params=pltpu.CompilerParams(needs_layout_passes=False),
    )
    def k(ctx_hbm, emb_hbm, out_hbm, idx_v, win_v, acc_v, sem):
        wid = lax.axis_index("s") * NC + lax.axis_index("c")
        base = wid * ROWS_PER_W
        pltpu.sync_copy(ctx_hbm.at[pl.ds(base, 16)], idx_v)
        idx_vec = idx_v[...]
        lanes = lax.iota(jnp.int32, 16)
        offs = []
        copies = []
        for j in range(ROWS_PER_W):
            ij = jnp.sum(idx_vec * (lanes == j).astype(jnp.int32))
            ws = pl.multiple_of((ij // 128) * 128, 128)
            offs.append(ij - ws)
            copies.append(
                pltpu.async_copy(
                    emb_hbm.at[:, pl.ds(ws, 128)], win_v.at[j], sem
                )
            )
        for cp in copies:
            cp.wait()
        scale = jnp.where(wid < ACTIVE_W, jnp.float32(1.0 / CTX), jnp.float32(0.0))
        for c in range(D // 16):
            row_idx = lax.iota(jnp.int32, 16) + (c * 16)
            s = None
            for j in range(ROWS_PER_W):
                col_idx = jnp.broadcast_to(offs[j], (16,))
                x = plsc.load_gather(win_v.at[j], [row_idx, col_idx])
                s = x if s is None else s + x
            acc_v[pl.ds(c * 16, 16)] = s * scale
        pltpu.sync_copy(acc_v, out_hbm.at[wid])

    return k(ctx_pad, in_emb_t)


def _tc_matvec(partials, out_emb_t):
    grid = pl.cdiv(VOCAB, BN)

    def body(part_ref, emb_ref, out_ref):
        v = jnp.sum(part_ref[...], axis=0).reshape(1, D)
        out_ref[...] = jax.lax.dot_general(
            v, emb_ref[...], (((1,), (0,)), ((), ())),
            preferred_element_type=jnp.float32).reshape(1, 1, BN)

    out2 = pl.pallas_call(
        body,
        grid=(grid,),
        in_specs=[
            pl.BlockSpec((NW, D), lambda i: (0, 0)),
            pl.BlockSpec((D, BN), lambda i: (0, i)),
        ],
        out_specs=pl.BlockSpec((1, 1, BN), lambda i: (i, 0, 0)),
        out_shape=jax.ShapeDtypeStruct((grid, 1, BN), jnp.float32),
    )(partials, out_emb_t)
    return out2.reshape(-1)[:VOCAB]


def kernel(context, in_emb, out_emb):
    ctx_pad = jnp.zeros((CTX_PAD,), jnp.int32).at[:CTX].set(context.astype(jnp.int32))
    partials = _sc_gather_mean(ctx_pad, in_emb.T)
    return _tc_matvec(partials, out_emb.T)


# speedup vs baseline: 47.8397x; 1.0333x over previous
import functools

import jax
import jax.numpy as jnp
from jax import lax
from jax.experimental import pallas as pl
from jax.experimental.pallas import tpu as pltpu
from jax.experimental.pallas import tpu_sc as plsc

VOCAB = 1000000
D = 64
CTX = 200

NC = 2
NS = 16
NW = NC * NS
ROWS_PER_W = 8
CTX_PAD = (NW + 1) * ROWS_PER_W
ACTIVE_W = CTX // ROWS_PER_W

BN = 32768


def _sc_gather_mean(ctx_pad, in_emb_t):
    mesh = plsc.VectorSubcoreMesh(core_axis_name="c", subcore_axis_name="s")

    @functools.partial(
        pl.kernel,
        out_type=jax.ShapeDtypeStruct((NW, D), jnp.float32),
        mesh=mesh,
        scratch_types=[
            pltpu.VMEM((16,), jnp.int32),
            pltpu.VMEM((ROWS_PER_W, D, 128), jnp.float32),
            pltpu.VMEM((D,), jnp.float32),
            pltpu.SemaphoreType.DMA,
        ],
        compiler_params=pltpu.CompilerParams(needs_layout_passes=False),
    )
    def k(ctx_hbm, emb_hbm, out_hbm, idx_v, win_v, acc_v, sem):
        wid = lax.axis_index("s") * NC + lax.axis_index("c")
        base = wid * ROWS_PER_W
        pltpu.sync_copy(ctx_hbm.at[pl.ds(base, 16)], idx_v)
        idx_vec = idx_v[...]
        lanes = lax.iota(jnp.int32, 16)
        offs = []
        copies = []
        for j in range(ROWS_PER_W):
            ij = jnp.sum(idx_vec * (lanes == j).astype(jnp.int32))
            ws = pl.multiple_of((ij // 128) * 128, 128)
            offs.append(ij - ws)
            copies.append(
                pltpu.async_copy(
                    emb_hbm.at[:, pl.ds(ws, 128)], win_v.at[j], sem
                )
            )
        for cp in copies:
            cp.wait()
        scale = jnp.where(wid < ACTIVE_W, jnp.float32(1.0 / CTX), jnp.float32(0.0))
        for c in range(D // 16):
            row_idx = lax.iota(jnp.int32, 16) + (c * 16)
            s = None
            for j in range(ROWS_PER_W):
                col_idx = jnp.broadcast_to(offs[j], (16,))
                x = plsc.load_gather(win_v.at[j], [row_idx, col_idx])
                s = x if s is None else s + x
            acc_v[pl.ds(c * 16, 16)] = s * scale
        pltpu.sync_copy(acc_v, out_hbm.at[wid])

    return k(ctx_pad, in_emb_t)


def _tc_matvec(partials, out_emb_t):
    grid = pl.cdiv(VOCAB, BN)

    def body(part_ref, emb_ref, out_ref):
        v = jnp.sum(part_ref[...], axis=0).reshape(1, D)
        out_ref[...] = jax.lax.dot_general(
            v, emb_ref[...], (((1,), (0,)), ((), ())),
            preferred_element_type=jnp.float32).reshape(BN)

    return pl.pallas_call(
        body,
        grid=(grid,),
        in_specs=[
            pl.BlockSpec((NW, D), lambda i: (0, 0)),
            pl.BlockSpec((D, BN), lambda i: (0, i)),
        ],
        out_specs=pl.BlockSpec((BN,), lambda i: (i,)),
        out_shape=jax.ShapeDtypeStruct((VOCAB,), jnp.float32),
    )(partials, out_emb_t)


def kernel(context, in_emb, out_emb):
    ctx_pad = jnp.zeros((CTX_PAD,), jnp.int32).at[:CTX].set(context.astype(jnp.int32))
    partials = _sc_gather_mean(ctx_pad, in_emb.T)
    return _tc_matvec(partials, out_emb.T)
